# Initial kernel scaffold; baseline (speedup 1.0000x reference)
import jax
import jax.numpy as jnp
from jax.experimental import pallas as pl


def kernel(x_user, x_food, edge_index_eats, edge_index_rev_eats, edge_label_index, Wp_user, bp_user, Wp_food, bp_food, W1e_rel, b1e_rel, W1e_root, W1r_rel, b1r_rel, W1r_root, W2e_rel, b2e_rel, W2e_root, W2r_rel, b2r_rel, W2r_root, Wd1, bd1, Wd2, bd2):
    raise NotImplementedError("write your pallas kernel here")


# speedup vs baseline: 1.2531x; 1.2531x over previous
import functools

import jax
import jax.numpy as jnp
from jax import lax
from jax.experimental import pallas as pl
from jax.experimental.pallas import tpu as pltpu
from jax.experimental.pallas import tpu_sc as plsc

N_USER = 10000
N_FOOD = 10000
E = 160000
L = 50000
D = 256
HALF = 128

NS = 16
NC = 2

E_PAD = 163840
L_PAD = 51200
ACC_ROWS = 10240
TRASH = 10000


def _offset_idx(idx_ref, rows, off):
    def body(t, carry):
        j = t // 8
        k = (t - j * 8) * 16
        idx_ref[j, pl.ds(k, 16)] = idx_ref[j, pl.ds(k, 16)] + off
        return carry
    lax.fori_loop(0, rows * 8, body, 0)


def _make_seg_sum(n_src, n_dst):
    CPT = E_PAD // (NS * 128)
    HCPT = CPT // 2
    NBUF = 2
    DHALF = n_dst // 2
    ACC = 5008
    LTRASH = DHALF
    ZR = 320
    mesh = plsc.VectorSubcoreMesh(core_axis_name="c", subcore_axis_name="s")

    @functools.partial(
        pl.kernel,
        out_type=jax.ShapeDtypeStruct((2 * n_dst, HALF), jnp.float32),
        mesh=mesh,
        scratch_types=[
            pltpu.VMEM((HCPT, 128), jnp.int32),
            pltpu.VMEM((HCPT, 128), jnp.int32),
            pltpu.VMEM((NBUF, 128, HALF), jnp.float32),
            pltpu.VMEM_SHARED((ACC, HALF), jnp.float32),
            pltpu.SemaphoreType.DMA,
        ],
    )
    def seg(x_hbm, src_hbm, dst_hbm, out_hbm, src_v, dst_v, rows_v, acc,
            sem):
        c = lax.axis_index("c")
        s = lax.axis_index("s")

        def zero_rows0(t, carry):
            i = t // 8
            k = (t - i * 8) * 16
            rows_v[0, i, pl.ds(k, 16)] = jnp.zeros((16,), jnp.float32)
            return carry

        for p in range(2):
            lax.fori_loop(0, 128 * 8, zero_rows0, 0)
            zst = jnp.where(s == NS - 1, ACC - ZR, s * ZR)
            zst = pl.multiple_of(zst, 8)
            pltpu.sync_copy(rows_v.at[0], acc.at[pl.ds(zst, 128)])
            pltpu.sync_copy(rows_v.at[0], acc.at[pl.ds(zst + 128, 128)])
            pltpu.sync_copy(rows_v.at[0, pl.ds(0, 64)],
                            acc.at[pl.ds(zst + 256, 64)])
            plsc.subcore_barrier()

            for h in range(2):
                base = s * CPT + h * HCPT
                pltpu.sync_copy(src_hbm.at[pl.ds(base, HCPT)], src_v)
                pltpu.sync_copy(dst_hbm.at[pl.ds(base, HCPT)], dst_v)

                def adjust_body(t, carry):
                    j = t // 8
                    k = (t - j * 8) * 16
                    src_v[j, pl.ds(k, 16)] = (
                        src_v[j, pl.ds(k, 16)] + c * n_src)
                    loc = dst_v[j, pl.ds(k, 16)] - p * DHALF
                    ok = (loc >= 0) & (loc < DHALF)
                    dst_v[j, pl.ds(k, 16)] = jnp.where(ok, loc, LTRASH)
                    return carry
                lax.fori_loop(0, HCPT * 8, adjust_body, 0)

                def group_body(g, carry):
                    descs = []
                    for b in range(NBUF):
                        descs.append(pltpu.async_copy(
                            x_hbm.at[src_v.at[g * NBUF + b]],
                            rows_v.at[b], sem))
                    for d in descs:
                        d.wait()
                    for b in range(NBUF):
                        pltpu.sync_copy(rows_v.at[b],
                                        acc.at[dst_v.at[g * NBUF + b]],
                                        add=True)
                    return carry
                lax.fori_loop(0, HCPT // NBUF, group_body, 0)

            plsc.subcore_barrier()

            start = jnp.where(s == NS - 1, DHALF - ZR, s * ZR)
            start = pl.multiple_of(start, 8)
            pltpu.sync_copy(
                acc.at[pl.ds(start, ZR)],
                out_hbm.at[pl.ds(c * n_dst + p * DHALF + start, ZR)])
            plsc.subcore_barrier()

    return seg


def _make_pair_gather(n_src, lp):
    CPT = lp // (NS * 128)
    NBUF = 5
    RPT = lp // NS
    mesh = plsc.VectorSubcoreMesh(core_axis_name="c", subcore_axis_name="s")

    @functools.partial(
        pl.kernel,
        out_type=[jax.ShapeDtypeStruct((2 * lp, HALF), jnp.float32),
                  jax.ShapeDtypeStruct((2 * lp, HALF), jnp.float32)],
        mesh=mesh,
        scratch_types=[
            pltpu.VMEM((CPT, 128), jnp.int32),
            pltpu.VMEM((CPT, 128), jnp.int32),
            pltpu.VMEM((NBUF, 128, HALF), jnp.float32),
            pltpu.SemaphoreType.DMA,
        ],
    )
    def gk(u_hbm, f_hbm, idxu_hbm, idxf_hbm, ue_hbm, fe_hbm,
           idxu_v, idxf_v, rows_v, sem):
        c = lax.axis_index("c")
        s = lax.axis_index("s")
        pltpu.sync_copy(idxu_hbm.at[s], idxu_v)
        pltpu.sync_copy(idxf_hbm.at[s], idxf_v)
        _offset_idx(idxu_v, CPT, c * n_src)
        _offset_idx(idxf_v, CPT, c * n_src)

        for idx_v, src_hbm, out_hbm in ((idxu_v, u_hbm, ue_hbm),
                                        (idxf_v, f_hbm, fe_hbm)):
            base = c * lp + s * RPT

            def gbody(g, carry, idx_v=idx_v, src_hbm=src_hbm,
                      out_hbm=out_hbm, base=base):
                descs = []
                for b in range(NBUF):
                    descs.append(pltpu.async_copy(
                        src_hbm.at[idx_v.at[g * NBUF + b]], rows_v.at[b],
                        sem))
                for d in descs:
                    d.wait()
                for b in range(NBUF):
                    pltpu.sync_copy(
                        rows_v.at[b],
                        out_hbm.at[pl.ds(base + (g * NBUF + b) * 128, 128)])
                return carry
            lax.fori_loop(0, CPT // NBUF, gbody, 0)

    return gk


_DN = (((1,), (1,)), ((), ()))


def _dot(a, b):
    return lax.dot_general(a, b, _DN, preferred_element_type=jnp.float32)


def _proj_body(xu, xf, wu, wf, bu, bf, hu, hf):
    hu[...] = _dot(xu[...], wu[...]) + bu[...]
    hf[...] = _dot(xf[...], wf[...]) + bf[...]


def _conv_body(ma0, ma1, xa0, xa1, wra, wta, ba,
               mb0, mb1, xb0, xb1, wrb, wtb, bb, oa, ob):
    ma = jnp.concatenate([ma0[...], ma1[...]], axis=1)
    xa = jnp.concatenate([xa0[...], xa1[...]], axis=1)
    oa[...] = jnp.maximum(_dot(ma, wra[...]) + _dot(xa, wta[...]) + ba[...],
                          0.0)
    mb = jnp.concatenate([mb0[...], mb1[...]], axis=1)
    xb = jnp.concatenate([xb0[...], xb1[...]], axis=1)
    ob[...] = jnp.maximum(_dot(mb, wrb[...]) + _dot(xb, wtb[...]) + bb[...],
                          0.0)


def _dec_body(ue0, ue1, fe0, fe1, wd1, bd1, wd2, bd2, out):
    comb = jnp.concatenate([ue0[...], ue1[...], fe0[...], fe1[...]], axis=1)
    h = jnp.maximum(_dot(comb, wd1[...]) + bd1[...], 0.0)
    o = _dot(h, wd2[...])[:, 0:1] + bd2[0, 0]
    out[...] = jax.nn.sigmoid(o)


def _split_specs(n, br, half_idx=None):
    nb = n // br
    s0 = pl.BlockSpec((br, HALF), lambda c, rb: (rb, 0))
    s1 = pl.BlockSpec((br, HALF), lambda c, rb: (nb + rb, 0))
    return s0, s1


def _proj(x_user, x_food, wu, wf, bu, bf):
    BR = 400
    nb = N_USER // BR
    grid = (2, nb)
    out_spec = pl.BlockSpec((BR, HALF), lambda c, rb: (c * nb + rb, 0))
    return pl.pallas_call(
        _proj_body,
        grid=grid,
        in_specs=[
            pl.BlockSpec((BR, D), lambda c, rb: (rb, 0)),
            pl.BlockSpec((BR, D), lambda c, rb: (rb, 0)),
            pl.BlockSpec((HALF, D), lambda c, rb: (c, 0)),
            pl.BlockSpec((HALF, D), lambda c, rb: (c, 0)),
            pl.BlockSpec((1, HALF), lambda c, rb: (0, c)),
            pl.BlockSpec((1, HALF), lambda c, rb: (0, c)),
        ],
        out_specs=[out_spec, out_spec],
        out_shape=[jax.ShapeDtypeStruct((2 * N_USER, HALF), jnp.float32),
                   jax.ShapeDtypeStruct((2 * N_FOOD, HALF), jnp.float32)],
    )(x_user, x_food, wu, wf, bu, bf)


def _conv(msg_a, x_a, wr_a, wt_a, b_a, msg_b, x_b, wr_b, wt_b, b_b):
    BR = 400
    nb = N_USER // BR
    grid = (2, nb)
    h0, h1 = _split_specs(N_USER, BR)
    wspec = pl.BlockSpec((HALF, D), lambda c, rb: (c, 0))
    bspec = pl.BlockSpec((1, HALF), lambda c, rb: (0, c))
    out_spec = pl.BlockSpec((BR, HALF), lambda c, rb: (c * nb + rb, 0))
    return pl.pallas_call(
        _conv_body,
        grid=grid,
        in_specs=[h0, h1, h0, h1, wspec, wspec, bspec,
                  h0, h1, h0, h1, wspec, wspec, bspec],
        out_specs=[out_spec, out_spec],
        out_shape=[jax.ShapeDtypeStruct((2 * N_FOOD, HALF), jnp.float32),
                   jax.ShapeDtypeStruct((2 * N_USER, HALF), jnp.float32)],
    )(msg_a, msg_a, x_a, x_a, wr_a, wt_a, b_a,
      msg_b, msg_b, x_b, x_b, wr_b, wt_b, b_b)


def _decoder(ue, fe, wd1, bd1, wd2, bd2):
    BR = 512
    nb = L_PAD // BR
    h0 = pl.BlockSpec((BR, HALF), lambda rb: (rb, 0))
    h1 = pl.BlockSpec((BR, HALF), lambda rb: (nb + rb, 0))
    return pl.pallas_call(
        _dec_body,
        grid=(nb,),
        in_specs=[h0, h1, h0, h1,
                  pl.BlockSpec((D, 2 * D), lambda rb: (0, 0)),
                  pl.BlockSpec((1, D), lambda rb: (0, 0)),
                  pl.BlockSpec((HALF, D), lambda rb: (0, 0)),
                  pl.BlockSpec(memory_space=pltpu.SMEM)],
        out_specs=pl.BlockSpec((BR, 1), lambda rb: (rb, 0)),
        out_shape=jax.ShapeDtypeStruct((L_PAD, 1), jnp.float32),
    )(ue, ue, fe, fe, wd1, bd1, wd2, bd2)


def _pad_edges(ei, total):
    src = ei[0].astype(jnp.int32)
    dst = ei[1].astype(jnp.int32)
    pad = total - src.shape[0]
    src_p = jnp.concatenate([src, jnp.zeros((pad,), jnp.int32)])
    dst_p = jnp.concatenate([dst, jnp.full((pad,), TRASH, jnp.int32)])
    return src_p.reshape(-1, 128), dst_p.reshape(-1, 128)


def kernel(x_user, x_food, edge_index_eats, edge_index_rev_eats,
           edge_label_index, Wp_user, bp_user, Wp_food, bp_food,
           W1e_rel, b1e_rel, W1e_root, W1r_rel, b1r_rel, W1r_root,
           W2e_rel, b2e_rel, W2e_root, W2r_rel, b2r_rel, W2r_root,
           Wd1, bd1, Wd2, bd2):
    src_e, dst_e = _pad_edges(edge_index_eats, E_PAD)
    src_r, dst_r = _pad_edges(edge_index_rev_eats, E_PAD)
    lpad = L_PAD - L
    idx_u = jnp.concatenate([edge_label_index[0].astype(jnp.int32),
                             jnp.zeros((lpad,), jnp.int32)]).reshape(NS, -1, 128)
    idx_f = jnp.concatenate([edge_label_index[1].astype(jnp.int32),
                             jnp.zeros((lpad,), jnp.int32)]).reshape(NS, -1, 128)

    hu, hf = _proj(x_user, x_food, Wp_user, Wp_food,
                   bp_user.reshape(1, D), bp_food.reshape(1, D))

    seg_uf = _make_seg_sum(N_USER, N_FOOD)
    seg_fu = _make_seg_sum(N_FOOD, N_USER)
    msg_f = seg_uf(hu, src_e, dst_e)
    msg_u = seg_fu(hf, src_r, dst_r)
    f1, u1 = _conv(msg_f, hf, W1e_rel, W1e_root, b1e_rel.reshape(1, D),
                   msg_u, hu, W1r_rel, W1r_root, b1r_rel.reshape(1, D))

    msg_f2 = seg_uf(u1, src_e, dst_e)
    msg_u2 = seg_fu(f1, src_r, dst_r)
    f2, u2 = _conv(msg_f2, f1, W2e_rel, W2e_root, b2e_rel.reshape(1, D),
                   msg_u2, u1, W2r_rel, W2r_root, b2r_rel.reshape(1, D))

    gk = _make_pair_gather(N_USER, L_PAD)
    ue, fe = gk(u2, f2, idx_u, idx_f)
    wd2p = jnp.zeros((HALF, D), jnp.float32).at[0].set(Wd2.reshape(D))
    out = _decoder(ue, fe, Wd1, bd1.reshape(1, D),
                   wd2p, bd2.reshape(1, 1))
    return out[:L, 0]


# speedup vs baseline: 1.3797x; 1.1010x over previous
import functools

import jax
import jax.numpy as jnp
from jax import lax
from jax.experimental import pallas as pl
from jax.experimental.pallas import tpu as pltpu
from jax.experimental.pallas import tpu_sc as plsc

N_USER = 10000
N_FOOD = 10000
E = 160000
L = 50000
D = 256
HALF = 128

NS = 16
NC = 2

E_PAD = 163840
L_PAD = 51200
ACC_ROWS = 10240
TRASH = 10000


def _offset_idx(idx_ref, rows, off):
    def body(t, carry):
        j = t // 8
        k = (t - j * 8) * 16
        idx_ref[j, pl.ds(k, 16)] = idx_ref[j, pl.ds(k, 16)] + off
        return carry
    lax.fori_loop(0, rows * 8, body, 0)


def _make_seg_sum(n_src, n_dst):
    CPT = E_PAD // (NS * 128)
    HCPT = CPT // 2
    NBUF = 2
    DHALF = n_dst // 2
    ACC = 5008
    LTRASH = DHALF
    ZR = 320
    mesh = plsc.VectorSubcoreMesh(core_axis_name="c", subcore_axis_name="s")

    @functools.partial(
        pl.kernel,
        out_type=jax.ShapeDtypeStruct((2 * n_dst, HALF), jnp.float32),
        mesh=mesh,
        scratch_types=[
            pltpu.VMEM((HCPT, 128), jnp.int32),
            pltpu.VMEM((HCPT, 128), jnp.int32),
            pltpu.VMEM((NBUF, 128, HALF), jnp.float32),
            pltpu.VMEM_SHARED((ACC, HALF), jnp.float32),
            pltpu.SemaphoreType.DMA,
            pltpu.SemaphoreType.DMA,
        ],
    )
    def seg(x_hbm, src_hbm, dst_hbm, out_hbm, src_v, dst_v, rows_v, acc,
            sem, ssem):
        c = lax.axis_index("c")
        s = lax.axis_index("s")

        def zero_rows0(t, carry):
            i = t // 8
            k = (t - i * 8) * 16
            rows_v[0, i, pl.ds(k, 16)] = jnp.zeros((16,), jnp.float32)
            return carry

        for p in range(2):
            lax.fori_loop(0, 128 * 8, zero_rows0, 0)
            zst = jnp.where(s == NS - 1, ACC - ZR, s * ZR)
            zst = pl.multiple_of(zst, 8)
            pltpu.sync_copy(rows_v.at[0], acc.at[pl.ds(zst, 128)])
            pltpu.sync_copy(rows_v.at[0], acc.at[pl.ds(zst + 128, 128)])
            pltpu.sync_copy(rows_v.at[0, pl.ds(0, 64)],
                            acc.at[pl.ds(zst + 256, 64)])
            plsc.subcore_barrier()

            for h in range(2):
                base = s * CPT + h * HCPT
                pltpu.sync_copy(src_hbm.at[pl.ds(base, HCPT)], src_v)
                pltpu.sync_copy(dst_hbm.at[pl.ds(base, HCPT)], dst_v)

                def adjust_body(t, carry):
                    j = t // 8
                    k = (t - j * 8) * 16
                    src_v[j, pl.ds(k, 16)] = (
                        src_v[j, pl.ds(k, 16)] + c * n_src)
                    loc = dst_v[j, pl.ds(k, 16)] - p * DHALF
                    ok = (loc >= 0) & (loc < DHALF)
                    dst_v[j, pl.ds(k, 16)] = jnp.where(ok, loc, LTRASH)
                    return carry
                lax.fori_loop(0, HCPT * 8, adjust_body, 0)

                gds = [pltpu.async_copy(x_hbm.at[src_v.at[b]],
                                        rows_v.at[b], sem)
                       for b in range(NBUF)]
                for b in range(NBUF):
                    gds[b].wait()
                    pltpu.async_copy(rows_v.at[b], acc.at[dst_v.at[b]],
                                     ssem, add=True)

                def group_body(g, carry):
                    gds = []
                    for b in range(NBUF):
                        pltpu.make_async_copy(
                            rows_v.at[b], acc.at[dst_v.at[0]], ssem).wait()
                        gds.append(pltpu.async_copy(
                            x_hbm.at[src_v.at[g * NBUF + b]],
                            rows_v.at[b], sem))
                    for b in range(NBUF):
                        gds[b].wait()
                        pltpu.async_copy(rows_v.at[b],
                                         acc.at[dst_v.at[g * NBUF + b]],
                                         ssem, add=True)
                    return carry
                lax.fori_loop(1, HCPT // NBUF, group_body, 0)
                for b in range(NBUF):
                    pltpu.make_async_copy(
                        rows_v.at[b], acc.at[dst_v.at[0]], ssem).wait()

            plsc.subcore_barrier()

            start = jnp.where(s == NS - 1, DHALF - ZR, s * ZR)
            start = pl.multiple_of(start, 8)
            pltpu.sync_copy(
                acc.at[pl.ds(start, ZR)],
                out_hbm.at[pl.ds(c * n_dst + p * DHALF + start, ZR)])
            plsc.subcore_barrier()

    return seg


def _make_pair_gather(n_src, lp):
    CPT = lp // (NS * 128)
    NBUF = 5
    RPT = lp // NS
    mesh = plsc.VectorSubcoreMesh(core_axis_name="c", subcore_axis_name="s")

    @functools.partial(
        pl.kernel,
        out_type=[jax.ShapeDtypeStruct((2 * lp, HALF), jnp.float32),
                  jax.ShapeDtypeStruct((2 * lp, HALF), jnp.float32)],
        mesh=mesh,
        scratch_types=[
            pltpu.VMEM((CPT, 128), jnp.int32),
            pltpu.VMEM((CPT, 128), jnp.int32),
            pltpu.VMEM((NBUF, 128, HALF), jnp.float32),
            pltpu.SemaphoreType.DMA,
        ],
    )
    def gk(u_hbm, f_hbm, idxu_hbm, idxf_hbm, ue_hbm, fe_hbm,
           idxu_v, idxf_v, rows_v, sem):
        c = lax.axis_index("c")
        s = lax.axis_index("s")
        pltpu.sync_copy(idxu_hbm.at[s], idxu_v)
        pltpu.sync_copy(idxf_hbm.at[s], idxf_v)
        _offset_idx(idxu_v, CPT, c * n_src)
        _offset_idx(idxf_v, CPT, c * n_src)

        for idx_v, src_hbm, out_hbm in ((idxu_v, u_hbm, ue_hbm),
                                        (idxf_v, f_hbm, fe_hbm)):
            base = c * lp + s * RPT

            def gbody(g, carry, idx_v=idx_v, src_hbm=src_hbm,
                      out_hbm=out_hbm, base=base):
                descs = []
                for b in range(NBUF):
                    descs.append(pltpu.async_copy(
                        src_hbm.at[idx_v.at[g * NBUF + b]], rows_v.at[b],
                        sem))
                for d in descs:
                    d.wait()
                for b in range(NBUF):
                    pltpu.sync_copy(
                        rows_v.at[b],
                        out_hbm.at[pl.ds(base + (g * NBUF + b) * 128, 128)])
                return carry
            lax.fori_loop(0, CPT // NBUF, gbody, 0)

    return gk


_DN = (((1,), (1,)), ((), ()))


def _dot(a, b):
    return lax.dot_general(a, b, _DN, preferred_element_type=jnp.float32)


def _proj_body(xu, xf, wu, wf, bu, bf, hu, hf):
    hu[...] = _dot(xu[...], wu[...]) + bu[...]
    hf[...] = _dot(xf[...], wf[...]) + bf[...]


def _conv_body(ma0, ma1, xa0, xa1, wra, wta, ba,
               mb0, mb1, xb0, xb1, wrb, wtb, bb, oa, ob):
    ma = jnp.concatenate([ma0[...], ma1[...]], axis=1)
    xa = jnp.concatenate([xa0[...], xa1[...]], axis=1)
    oa[...] = jnp.maximum(_dot(ma, wra[...]) + _dot(xa, wta[...]) + ba[...],
                          0.0)
    mb = jnp.concatenate([mb0[...], mb1[...]], axis=1)
    xb = jnp.concatenate([xb0[...], xb1[...]], axis=1)
    ob[...] = jnp.maximum(_dot(mb, wrb[...]) + _dot(xb, wtb[...]) + bb[...],
                          0.0)


def _dec_body(ue0, ue1, fe0, fe1, wd1, bd1, wd2, bd2, out):
    comb = jnp.concatenate([ue0[...], ue1[...], fe0[...], fe1[...]], axis=1)
    h = jnp.maximum(_dot(comb, wd1[...]) + bd1[...], 0.0)
    o = _dot(h, wd2[...])[:, 0:1] + bd2[0, 0]
    out[...] = jax.nn.sigmoid(o)


def _split_specs(n, br, half_idx=None):
    nb = n // br
    s0 = pl.BlockSpec((br, HALF), lambda c, rb: (rb, 0))
    s1 = pl.BlockSpec((br, HALF), lambda c, rb: (nb + rb, 0))
    return s0, s1


def _proj(x_user, x_food, wu, wf, bu, bf):
    BR = 400
    nb = N_USER // BR
    grid = (2, nb)
    out_spec = pl.BlockSpec((BR, HALF), lambda c, rb: (c * nb + rb, 0))
    return pl.pallas_call(
        _proj_body,
        grid=grid,
        in_specs=[
            pl.BlockSpec((BR, D), lambda c, rb: (rb, 0)),
            pl.BlockSpec((BR, D), lambda c, rb: (rb, 0)),
            pl.BlockSpec((HALF, D), lambda c, rb: (c, 0)),
            pl.BlockSpec((HALF, D), lambda c, rb: (c, 0)),
            pl.BlockSpec((1, HALF), lambda c, rb: (0, c)),
            pl.BlockSpec((1, HALF), lambda c, rb: (0, c)),
        ],
        out_specs=[out_spec, out_spec],
        out_shape=[jax.ShapeDtypeStruct((2 * N_USER, HALF), jnp.float32),
                   jax.ShapeDtypeStruct((2 * N_FOOD, HALF), jnp.float32)],
    )(x_user, x_food, wu, wf, bu, bf)


def _conv(msg_a, x_a, wr_a, wt_a, b_a, msg_b, x_b, wr_b, wt_b, b_b):
    BR = 400
    nb = N_USER // BR
    grid = (2, nb)
    h0, h1 = _split_specs(N_USER, BR)
    wspec = pl.BlockSpec((HALF, D), lambda c, rb: (c, 0))
    bspec = pl.BlockSpec((1, HALF), lambda c, rb: (0, c))
    out_spec = pl.BlockSpec((BR, HALF), lambda c, rb: (c * nb + rb, 0))
    return pl.pallas_call(
        _conv_body,
        grid=grid,
        in_specs=[h0, h1, h0, h1, wspec, wspec, bspec,
                  h0, h1, h0, h1, wspec, wspec, bspec],
        out_specs=[out_spec, out_spec],
        out_shape=[jax.ShapeDtypeStruct((2 * N_FOOD, HALF), jnp.float32),
                   jax.ShapeDtypeStruct((2 * N_USER, HALF), jnp.float32)],
    )(msg_a, msg_a, x_a, x_a, wr_a, wt_a, b_a,
      msg_b, msg_b, x_b, x_b, wr_b, wt_b, b_b)


def _decoder(ue, fe, wd1, bd1, wd2, bd2):
    BR = 512
    nb = L_PAD // BR
    h0 = pl.BlockSpec((BR, HALF), lambda rb: (rb, 0))
    h1 = pl.BlockSpec((BR, HALF), lambda rb: (nb + rb, 0))
    return pl.pallas_call(
        _dec_body,
        grid=(nb,),
        in_specs=[h0, h1, h0, h1,
                  pl.BlockSpec((D, 2 * D), lambda rb: (0, 0)),
                  pl.BlockSpec((1, D), lambda rb: (0, 0)),
                  pl.BlockSpec((HALF, D), lambda rb: (0, 0)),
                  pl.BlockSpec(memory_space=pltpu.SMEM)],
        out_specs=pl.BlockSpec((BR, 1), lambda rb: (rb, 0)),
        out_shape=jax.ShapeDtypeStruct((L_PAD, 1), jnp.float32),
    )(ue, ue, fe, fe, wd1, bd1, wd2, bd2)


def _pad_edges(ei, total):
    src = ei[0].astype(jnp.int32)
    dst = ei[1].astype(jnp.int32)
    pad = total - src.shape[0]
    src_p = jnp.concatenate([src, jnp.zeros((pad,), jnp.int32)])
    dst_p = jnp.concatenate([dst, jnp.full((pad,), TRASH, jnp.int32)])
    return src_p.reshape(-1, 128), dst_p.reshape(-1, 128)


def kernel(x_user, x_food, edge_index_eats, edge_index_rev_eats,
           edge_label_index, Wp_user, bp_user, Wp_food, bp_food,
           W1e_rel, b1e_rel, W1e_root, W1r_rel, b1r_rel, W1r_root,
           W2e_rel, b2e_rel, W2e_root, W2r_rel, b2r_rel, W2r_root,
           Wd1, bd1, Wd2, bd2):
    src_e, dst_e = _pad_edges(edge_index_eats, E_PAD)
    src_r, dst_r = _pad_edges(edge_index_rev_eats, E_PAD)
    lpad = L_PAD - L
    idx_u = jnp.concatenate([edge_label_index[0].astype(jnp.int32),
                             jnp.zeros((lpad,), jnp.int32)]).reshape(NS, -1, 128)
    idx_f = jnp.concatenate([edge_label_index[1].astype(jnp.int32),
                             jnp.zeros((lpad,), jnp.int32)]).reshape(NS, -1, 128)

    hu, hf = _proj(x_user, x_food, Wp_user, Wp_food,
                   bp_user.reshape(1, D), bp_food.reshape(1, D))

    seg_uf = _make_seg_sum(N_USER, N_FOOD)
    seg_fu = _make_seg_sum(N_FOOD, N_USER)
    msg_f = seg_uf(hu, src_e, dst_e)
    msg_u = seg_fu(hf, src_r, dst_r)
    f1, u1 = _conv(msg_f, hf, W1e_rel, W1e_root, b1e_rel.reshape(1, D),
                   msg_u, hu, W1r_rel, W1r_root, b1r_rel.reshape(1, D))

    msg_f2 = seg_uf(u1, src_e, dst_e)
    msg_u2 = seg_fu(f1, src_r, dst_r)
    f2, u2 = _conv(msg_f2, f1, W2e_rel, W2e_root, b2e_rel.reshape(1, D),
                   msg_u2, u1, W2r_rel, W2r_root, b2r_rel.reshape(1, D))

    gk = _make_pair_gather(N_USER, L_PAD)
    ue, fe = gk(u2, f2, idx_u, idx_f)
    wd2p = jnp.zeros((HALF, D), jnp.float32).at[0].set(Wd2.reshape(D))
    out = _decoder(ue, fe, Wd1, bd1.reshape(1, D),
                   wd2p, bd2.reshape(1, 1))
    return out[:L, 0]
